# baseline (speedup 1.0000x reference)
import jax
import jax.numpy as jnp
from jax.experimental import pallas as pl
from jax.experimental.pallas import tpu as pltpu


def _mm_block(vb_ref, a_ref, o_ref, v_scr):
    @pl.when(pl.program_id(0) == 0)
    def _pack():
        v_scr[:, 0:16] = vb_ref[0]
        v_scr[:, 16:32] = vb_ref[1]

    o = jnp.dot(a_ref[...], v_scr[...], preferred_element_type=jnp.float32)
    o_ref[0] = o[:, 0:16]
    o_ref[1] = o[:, 16:32]


def kernel(matrix, matrix_batch):
    m, k = matrix.shape
    b, _, f = matrix_batch.shape
    n = b * f

    bm = 256
    return pl.pallas_call(
        _mm_block,
        grid=(m // bm,),
        in_specs=[
            pl.BlockSpec((b, k, f), lambda i: (0, 0, 0)),
            pl.BlockSpec((bm, k), lambda i: (i, 0)),
        ],
        out_specs=pl.BlockSpec((b, bm, f), lambda i: (0, i, 0)),
        out_shape=jax.ShapeDtypeStruct((b, m, f), jnp.float32),
        scratch_shapes=[pltpu.VMEM((k, n), jnp.float32)],
    )(matrix_batch, matrix)


# speedup vs baseline: 1.0403x; 1.0403x over previous
import jax
import jax.numpy as jnp
from jax.experimental import pallas as pl
from jax.experimental.pallas import tpu as pltpu

_BM = 256
_NBUF = 4


def _mm_manual(a_hbm, v_ref, o_ref, bufs, sems):
    steps = a_hbm.shape[0] // _BM

    def start_copy(slot, blk):
        pltpu.make_async_copy(
            a_hbm.at[pl.ds(blk * _BM, _BM), :], bufs.at[slot], sems.at[slot]
        ).start()

    for s in range(_NBUF):
        start_copy(s, s)

    v = v_ref[...]

    def step(i, carry):
        slot = jax.lax.rem(i, _NBUF)
        pltpu.make_async_copy(
            a_hbm.at[pl.ds(i * _BM, _BM), :], bufs.at[slot], sems.at[slot]
        ).wait()
        o_ref[pl.ds(i * _BM, _BM), :] = jnp.dot(
            bufs[slot], v, preferred_element_type=jnp.float32
        )

        @pl.when(i + _NBUF < steps)
        def _next():
            start_copy(slot, i + _NBUF)

        return carry

    jax.lax.fori_loop(0, steps, step, 0)


def kernel(matrix, matrix_batch):
    m, k = matrix.shape
    b, _, f = matrix_batch.shape
    n = b * f
    vectors = jnp.swapaxes(matrix_batch, 0, 1).reshape(k, n)

    out = pl.pallas_call(
        _mm_manual,
        in_specs=[
            pl.BlockSpec(memory_space=pltpu.MemorySpace.HBM),
            pl.BlockSpec(memory_space=pltpu.MemorySpace.VMEM),
        ],
        out_specs=pl.BlockSpec(memory_space=pltpu.MemorySpace.VMEM),
        out_shape=jax.ShapeDtypeStruct((m, n), jnp.float32),
        scratch_shapes=[
            pltpu.VMEM((_NBUF, _BM, k), jnp.float32),
            pltpu.SemaphoreType.DMA((_NBUF,)),
        ],
    )(matrix, vectors)

    return jnp.swapaxes(out.reshape(m, b, f), 0, 1)


# speedup vs baseline: 1.0998x; 1.0573x over previous
import jax
import jax.numpy as jnp
from jax.experimental import pallas as pl
from jax.experimental.pallas import tpu as pltpu

_BM = 256


def _mm(a_ref, v_ref, o_ref):
    o_ref[...] = jnp.dot(a_ref[...], v_ref[...], preferred_element_type=jnp.float32)


def kernel(matrix, matrix_batch):
    m, k = matrix.shape
    b, _, f = matrix_batch.shape
    n = b * f
    vectors = jnp.swapaxes(matrix_batch, 0, 1).reshape(k, n)

    out = pl.pallas_call(
        _mm,
        grid=(m // _BM,),
        in_specs=[
            pl.BlockSpec((_BM, k), lambda i: (i, 0)),
            pl.BlockSpec((k, n), lambda i: (0, 0)),
        ],
        out_specs=pl.BlockSpec((_BM, n), lambda i: (i, 0)),
        out_shape=jax.ShapeDtypeStruct((m, n), jnp.float32),
    )(matrix, vectors)

    return jnp.swapaxes(out.reshape(m, b, f), 0, 1)
